# Initial kernel scaffold; baseline (speedup 1.0000x reference)
import jax
import jax.numpy as jnp
from jax.experimental import pallas as pl


def kernel(x, edge_index, last_rej_rate, W1, b1, W2, b2, W3, b3):
    raise NotImplementedError("write your pallas kernel here")


# speedup vs baseline: 25.6974x; 25.6974x over previous
import functools

import jax
import jax.numpy as jnp
from jax import lax
from jax.experimental import pallas as pl
from jax.experimental.pallas import tpu as pltpu
from jax.experimental.pallas import tpu_sc as plsc

N = 10000
E = 320000
D_IN, D_HID, D_OUT = 128, 64, 32

NC, NS = 2, 16
NW = NC * NS
EPT = E // NW
CHUNK = 125
CHUNKS = EPT // CHUNK
NP = 10240
RPT = NP // NS
DEGC = 16


def _mesh():
    return plsc.VectorSubcoreMesh(
        core_axis_name="c", subcore_axis_name="s", num_cores=NC, num_subcores=NS
    )


def _make_deg():
    @functools.partial(
        pl.kernel,
        out_type=jax.ShapeDtypeStruct((NC, NP, DEGC), jnp.float32),
        mesh=_mesh(),
        compiler_params=pltpu.CompilerParams(use_tc_tiling_on_sc=False),
        scratch_types=[
            pltpu.VMEM((CHUNKS, CHUNK), jnp.int32),
            pltpu.VMEM((RPT, DEGC), jnp.float32),
            pltpu.VMEM((CHUNK, DEGC), jnp.float32),
            pltpu.VMEM_SHARED((NP, DEGC), jnp.float32),
        ],
    )
    def deg(dst_hbm, out_hbm, dst_v, obuf, ones_v, acc):
        c = lax.axis_index("c")
        s = lax.axis_index("s")
        wid = c * NS + s
        pltpu.sync_copy(dst_hbm.at[wid], dst_v)

        def fill_obuf(i, carry):
            obuf[i] = jnp.ones((DEGC,), jnp.float32)
            return carry

        lax.fori_loop(0, RPT, fill_obuf, 0)

        def fill_ones(i, carry):
            ones_v[i] = jnp.ones((DEGC,), jnp.float32)
            return carry

        lax.fori_loop(0, CHUNK, fill_ones, 0)

        r0 = s * RPT
        pltpu.sync_copy(obuf, acc.at[pl.ds(r0, RPT)])
        plsc.subcore_barrier()

        def body(j, carry):
            pltpu.sync_copy(ones_v, acc.at[dst_v.at[j]], add=True)
            return carry

        lax.fori_loop(0, CHUNKS, body, 0)
        plsc.subcore_barrier()
        pltpu.sync_copy(acc.at[pl.ds(r0, RPT)], obuf)
        pltpu.sync_copy(obuf, out_hbm.at[c, pl.ds(r0, RPT)])

    return deg


def _make_agg(D):
    @functools.partial(
        pl.kernel,
        out_type=jax.ShapeDtypeStruct((NC, NP, D), jnp.float32),
        mesh=_mesh(),
        compiler_params=pltpu.CompilerParams(use_tc_tiling_on_sc=False),
        scratch_types=[
            pltpu.VMEM((CHUNKS, CHUNK), jnp.int32),
            pltpu.VMEM((CHUNKS, CHUNK), jnp.int32),
            pltpu.VMEM((CHUNK, D), jnp.float32),
            pltpu.VMEM((RPT, D), jnp.float32),
            pltpu.VMEM_SHARED((NP, D), jnp.float32),
        ],
    )
    def agg(src_hbm, dst_hbm, table_hbm, out_hbm, src_v, dst_v, gbuf, bounce, acc):
        c = lax.axis_index("c")
        s = lax.axis_index("s")
        wid = c * NS + s
        pltpu.sync_copy(src_hbm.at[wid], src_v)
        pltpu.sync_copy(dst_hbm.at[wid], dst_v)
        r0 = s * RPT
        pltpu.sync_copy(table_hbm.at[pl.ds(r0, RPT)], bounce)
        pltpu.sync_copy(bounce, acc.at[pl.ds(r0, RPT)])
        plsc.subcore_barrier()

        def body(j, carry):
            pltpu.sync_copy(table_hbm.at[src_v.at[j]], gbuf)
            pltpu.sync_copy(gbuf, acc.at[dst_v.at[j]], add=True)
            return carry

        lax.fori_loop(0, CHUNKS, body, 0)
        plsc.subcore_barrier()
        pltpu.sync_copy(acc.at[pl.ds(r0, RPT)], bounce)
        pltpu.sync_copy(bounce, out_hbm.at[c, pl.ds(r0, RPT)])

    return agg


@functools.lru_cache(maxsize=None)
def _get_deg():
    return _make_deg()


@functools.lru_cache(maxsize=None)
def _get_agg(D):
    return _make_agg(D)


def _tc1_body(x_ref, w_ref, d0_ref, d1_ref, dinv_ref, ht_ref):
    deg = d0_ref[:, 0:1] + d1_ref[:, 0:1] - 1.0
    dinv = lax.rsqrt(deg)
    dinv_ref[...] = dinv
    y = jnp.dot(x_ref[...], w_ref[...], preferred_element_type=jnp.float32)
    ht_ref[...] = dinv * y


def _tc1(x, W1, d):
    return pl.pallas_call(
        _tc1_body,
        out_shape=[
            jax.ShapeDtypeStruct((NP, 1), jnp.float32),
            jax.ShapeDtypeStruct((NP, D_HID), jnp.float32),
        ],
    )(x, W1, d[0], d[1])


def _comb_body(p_ref, ht_ref, dinv_ref, b_ref, w_ref, out_ref):
    dinv = dinv_ref[...]
    t = p_ref[0] + p_ref[1] - ht_ref[...]
    h = jnp.maximum(dinv * t + b_ref[...], 0.0)
    out_ref[...] = dinv * jnp.dot(
        h, w_ref[...], preferred_element_type=jnp.float32
    )


def _comb(p, ht, dinv, b, W, D_next):
    return pl.pallas_call(
        _comb_body,
        out_shape=jax.ShapeDtypeStruct((NP, D_next), jnp.float32),
    )(p, ht, dinv, b, W)


def _final_body(p_ref, ht_ref, dinv_ref, b_ref, out_ref):
    dinv = dinv_ref[...]
    t = p_ref[0] + p_ref[1] - ht_ref[...]
    h = jnp.maximum(dinv * t + b_ref[...], 0.0)
    out_ref[...] = jnp.sum(h[0:N], axis=0, keepdims=True) * (1.0 / N)


def _final(p, ht, dinv, b):
    return pl.pallas_call(
        _final_body,
        out_shape=jax.ShapeDtypeStruct((1, D_OUT), jnp.float32),
    )(p, ht, dinv, b)


def kernel(x, edge_index, last_rej_rate, W1, b1, W2, b2, W3, b3):
    srcm = edge_index[0].reshape(NW, CHUNKS, CHUNK)
    dstm = edge_index[1].reshape(NW, CHUNKS, CHUNK)
    x = jnp.concatenate([x, jnp.zeros((NP - N, D_IN), jnp.float32)], axis=0)
    d = _get_deg()(dstm)
    dinv, ht1 = _tc1(x, W1, d)
    p1 = _get_agg(D_HID)(srcm, dstm, ht1)
    ht2 = _comb(p1, ht1, dinv, b1.reshape(1, -1), W2, D_HID)
    p2 = _get_agg(D_HID)(srcm, dstm, ht2)
    ht3 = _comb(p2, ht2, dinv, b2.reshape(1, -1), W3, D_OUT)
    p3 = _get_agg(D_OUT)(srcm, dstm, ht3)
    pooled = _final(p3, ht3, dinv, b3.reshape(1, -1))
    rej = jnp.reshape(last_rej_rate, (1, 1)).astype(jnp.float32)
    return jnp.concatenate([pooled, rej], axis=-1)


# speedup vs baseline: 30.4372x; 1.1844x over previous
import functools

import jax
import jax.numpy as jnp
from jax import lax
from jax.experimental import pallas as pl
from jax.experimental.pallas import tpu as pltpu
from jax.experimental.pallas import tpu_sc as plsc

N = 10000
E = 320000
D_IN, D_HID, D_OUT = 128, 64, 32

NC, NS = 2, 16
NW = NC * NS
EPT = E // NW
CHUNK = 125
CHUNKS = EPT // CHUNK
NP = 10240
RPT = NP // NS
DEGC = 16


def _mesh():
    return plsc.VectorSubcoreMesh(
        core_axis_name="c", subcore_axis_name="s", num_cores=NC, num_subcores=NS
    )


def _make_deg():
    @functools.partial(
        pl.kernel,
        out_type=jax.ShapeDtypeStruct((NC, NP, DEGC), jnp.float32),
        mesh=_mesh(),
        compiler_params=pltpu.CompilerParams(use_tc_tiling_on_sc=False),
        scratch_types=[
            pltpu.VMEM((CHUNKS, CHUNK), jnp.int32),
            pltpu.VMEM((RPT, DEGC), jnp.float32),
            pltpu.VMEM((CHUNK, DEGC), jnp.float32),
            pltpu.VMEM_SHARED((NP, DEGC), jnp.float32),
        ],
    )
    def deg(dst_hbm, out_hbm, dst_v, obuf, ones_v, acc):
        c = lax.axis_index("c")
        s = lax.axis_index("s")
        wid = c * NS + s
        pltpu.sync_copy(dst_hbm.at[wid], dst_v)

        def fill_obuf(i, carry):
            obuf[i] = jnp.ones((DEGC,), jnp.float32)
            return carry

        lax.fori_loop(0, RPT, fill_obuf, 0)

        def fill_ones(i, carry):
            ones_v[i] = jnp.ones((DEGC,), jnp.float32)
            return carry

        lax.fori_loop(0, CHUNK, fill_ones, 0)

        r0 = s * RPT
        pltpu.sync_copy(obuf, acc.at[pl.ds(r0, RPT)])
        plsc.subcore_barrier()

        def body(j, carry):
            pltpu.sync_copy(ones_v, acc.at[dst_v.at[j]], add=True)
            return carry

        lax.fori_loop(0, CHUNKS, body, 0)
        plsc.subcore_barrier()
        pltpu.sync_copy(acc.at[pl.ds(r0, RPT)], obuf)
        pltpu.sync_copy(obuf, out_hbm.at[c, pl.ds(r0, RPT)])

    return deg


NBUF = 4
RING_STEPS = CHUNKS // NBUF


def _make_agg(D):
    @functools.partial(
        pl.kernel,
        out_type=jax.ShapeDtypeStruct((NC, NP, D), jnp.float32),
        mesh=_mesh(),
        compiler_params=pltpu.CompilerParams(use_tc_tiling_on_sc=False),
        scratch_types=[
            pltpu.VMEM((CHUNKS, CHUNK), jnp.int32),
            pltpu.VMEM((CHUNKS, CHUNK), jnp.int32),
            pltpu.VMEM((NBUF, CHUNK, D), jnp.float32),
            pltpu.VMEM_SHARED((NP, D), jnp.float32),
            pltpu.SemaphoreType.DMA((NBUF,)),
            pltpu.SemaphoreType.DMA((NBUF,)),
        ],
    )
    def agg(src_hbm, dst_hbm, table_hbm, out_hbm, src_v, dst_v, gbuf,
            acc, gsem, ssem):
        c = lax.axis_index("c")
        s = lax.axis_index("s")
        wid = c * NS + s
        pltpu.sync_copy(src_hbm.at[wid], src_v)
        pltpu.sync_copy(dst_hbm.at[wid], dst_v)

        def g_start(j, b):
            pltpu.async_copy(table_hbm.at[src_v.at[j]], gbuf.at[b], gsem.at[b])

        def g_wait(j, b):
            pltpu.make_async_copy(
                table_hbm.at[src_v.at[j]], gbuf.at[b], gsem.at[b]).wait()

        def s_start(j, b):
            pltpu.async_copy(gbuf.at[b], acc.at[dst_v.at[j]], ssem.at[b],
                             add=True)

        def s_wait(j, b):
            pltpu.make_async_copy(
                gbuf.at[b], acc.at[dst_v.at[j]], ssem.at[b]).wait()

        g_start(0, 0)
        r0 = s * RPT
        pltpu.sync_copy(table_hbm.at[pl.ds(r0, RPT)], acc.at[pl.ds(r0, RPT)])
        plsc.subcore_barrier()

        def body(g, carry):
            j0 = g * NBUF
            for b in range(NBUF):
                j = j0 + b
                nb = (b + 1) % NBUF
                g_wait(j, b)
                if b == NBUF - 1:
                    s_wait(j - (NBUF - 1), nb)
                else:
                    @pl.when(g > 0)
                    def _():
                        s_wait(j - (NBUF - 1), nb)
                s_start(j, b)
                if b == NBUF - 1:
                    @pl.when(g < RING_STEPS - 1)
                    def _():
                        g_start(j + 1, nb)
                else:
                    g_start(j + 1, nb)
            return carry

        lax.fori_loop(0, RING_STEPS, body, 0)
        for b in range(1, NBUF):
            s_wait(CHUNKS - NBUF + b, b)
        plsc.subcore_barrier()
        pltpu.sync_copy(acc.at[pl.ds(r0, RPT)], out_hbm.at[c, pl.ds(r0, RPT)])

    return agg


@functools.lru_cache(maxsize=None)
def _get_deg():
    return _make_deg()


@functools.lru_cache(maxsize=None)
def _get_agg(D):
    return _make_agg(D)


def _tc1_body(x_ref, w_ref, d0_ref, d1_ref, dinv_ref, ht_ref):
    deg = d0_ref[:, 0:1] + d1_ref[:, 0:1] - 1.0
    dinv = lax.rsqrt(deg)
    dinv_ref[...] = dinv
    y = jnp.dot(x_ref[...], w_ref[...], preferred_element_type=jnp.float32)
    ht_ref[...] = dinv * y


def _tc1(x, W1, d):
    return pl.pallas_call(
        _tc1_body,
        out_shape=[
            jax.ShapeDtypeStruct((NP, 1), jnp.float32),
            jax.ShapeDtypeStruct((NP, D_HID), jnp.float32),
        ],
    )(x, W1, d[0], d[1])


def _comb_body(p_ref, ht_ref, dinv_ref, b_ref, w_ref, out_ref):
    dinv = dinv_ref[...]
    t = p_ref[0] + p_ref[1] - ht_ref[...]
    h = jnp.maximum(dinv * t + b_ref[...], 0.0)
    out_ref[...] = dinv * jnp.dot(
        h, w_ref[...], preferred_element_type=jnp.float32
    )


def _comb(p, ht, dinv, b, W, D_next):
    return pl.pallas_call(
        _comb_body,
        out_shape=jax.ShapeDtypeStruct((NP, D_next), jnp.float32),
    )(p, ht, dinv, b, W)


def _final_body(p_ref, ht_ref, dinv_ref, b_ref, out_ref):
    dinv = dinv_ref[...]
    t = p_ref[0] + p_ref[1] - ht_ref[...]
    h = jnp.maximum(dinv * t + b_ref[...], 0.0)
    out_ref[...] = jnp.sum(h[0:N], axis=0, keepdims=True) * (1.0 / N)


def _final(p, ht, dinv, b):
    return pl.pallas_call(
        _final_body,
        out_shape=jax.ShapeDtypeStruct((1, D_OUT), jnp.float32),
    )(p, ht, dinv, b)


def kernel(x, edge_index, last_rej_rate, W1, b1, W2, b2, W3, b3):
    srcm = edge_index[0].reshape(NW, CHUNKS, CHUNK)
    dstm = edge_index[1].reshape(NW, CHUNKS, CHUNK)
    x = jnp.concatenate([x, jnp.zeros((NP - N, D_IN), jnp.float32)], axis=0)
    d = _get_deg()(dstm)
    dinv, ht1 = _tc1(x, W1, d)
    p1 = _get_agg(D_HID)(srcm, dstm, ht1)
    ht2 = _comb(p1, ht1, dinv, b1.reshape(1, -1), W2, D_HID)
    p2 = _get_agg(D_HID)(srcm, dstm, ht2)
    ht3 = _comb(p2, ht2, dinv, b2.reshape(1, -1), W3, D_OUT)
    p3 = _get_agg(D_OUT)(srcm, dstm, ht3)
    pooled = _final(p3, ht3, dinv, b3.reshape(1, -1))
    rej = jnp.reshape(last_rej_rate, (1, 1)).astype(jnp.float32)
    return jnp.concatenate([pooled, rej], axis=-1)
